# baseline (speedup 1.0000x reference)
import functools

import jax
import jax.numpy as jnp
from jax import lax
from jax.experimental import pallas as pl
from jax.experimental.pallas import tpu as pltpu
from jax.experimental.pallas import tpu_sc as plsc

V, D, O = 1000000, 64, 256
B, L = 4096, 200

NC, NS = 2, 16
NW = NC * NS
EPW = B // NW
HALF = L // 2


def _make_sc_sum():
    mesh = plsc.VectorSubcoreMesh(core_axis_name="c", subcore_axis_name="s")

    @functools.partial(
        pl.kernel,
        mesh=mesh,
        compiler_params=pltpu.CompilerParams(use_tc_tiling_on_sc=False),
        out_type=jax.ShapeDtypeStruct((B, D), jnp.float32),
        scratch_types=[
            pltpu.VMEM((2 * EPW, HALF), jnp.int32),
            pltpu.VMEM((HALF, D), jnp.float32),
            pltpu.VMEM((HALF, D), jnp.float32),
            pltpu.VMEM((HALF, D), jnp.float32),
            pltpu.VMEM((HALF, D), jnp.float32),
            pltpu.VMEM((EPW, D), jnp.float32),
            pltpu.SemaphoreType.DMA,
            pltpu.SemaphoreType.DMA,
        ],
    )
    def sc_sum(tok_hbm, table_hbm, out_hbm, tok_v, a0_v, a1_v, b0_v, b1_v,
               out_v, sem_a, sem_b):
        wid = lax.axis_index("s") * NC + lax.axis_index("c")
        pltpu.sync_copy(tok_hbm.at[pl.ds(wid * (2 * EPW), 2 * EPW)], tok_v)

        def issue(e, r0, r1, sem):
            pltpu.async_copy(table_hbm.at[tok_v.at[2 * e]], r0, sem)
            pltpu.async_copy(table_hbm.at[tok_v.at[2 * e + 1]], r1, sem)

        def drain(r0, r1, sem):
            pltpu.make_async_copy(table_hbm.at[tok_v.at[0]], r0, sem).wait()
            pltpu.make_async_copy(table_hbm.at[tok_v.at[0]], r1, sem).wait()

        def sumbuf(e, r0, r1):
            z = jnp.zeros((16,), jnp.float32)

            def rbody(i, acc):
                a0, a1, a2, a3 = acc
                r = 2 * i
                a0 = a0 + r0[r, pl.ds(0, 16)] + r1[r, pl.ds(0, 16)]
                a1 = a1 + r0[r, pl.ds(16, 16)] + r1[r, pl.ds(16, 16)]
                a2 = a2 + r0[r, pl.ds(32, 16)] + r1[r, pl.ds(32, 16)]
                a3 = a3 + r0[r, pl.ds(48, 16)] + r1[r, pl.ds(48, 16)]
                s = r + 1
                a0 = a0 + r0[s, pl.ds(0, 16)] + r1[s, pl.ds(0, 16)]
                a1 = a1 + r0[s, pl.ds(16, 16)] + r1[s, pl.ds(16, 16)]
                a2 = a2 + r0[s, pl.ds(32, 16)] + r1[s, pl.ds(32, 16)]
                a3 = a3 + r0[s, pl.ds(48, 16)] + r1[s, pl.ds(48, 16)]
                return (a0, a1, a2, a3)

            a0, a1, a2, a3 = lax.fori_loop(0, HALF // 2, rbody, (z, z, z, z))
            out_v[e, pl.ds(0, 16)] = a0
            out_v[e, pl.ds(16, 16)] = a1
            out_v[e, pl.ds(32, 16)] = a2
            out_v[e, pl.ds(48, 16)] = a3

        issue(0, a0_v, a1_v, sem_a)

        def pair(i, carry):
            issue(2 * i + 1, b0_v, b1_v, sem_b)
            drain(a0_v, a1_v, sem_a)
            sumbuf(2 * i, a0_v, a1_v)

            @pl.when(i < EPW // 2 - 1)
            def _():
                issue(2 * i + 2, a0_v, a1_v, sem_a)

            drain(b0_v, b1_v, sem_b)
            sumbuf(2 * i + 1, b0_v, b1_v)
            return carry

        lax.fori_loop(0, EPW // 2, pair, 0)
        pltpu.sync_copy(out_v, out_hbm.at[pl.ds(wid * EPW, EPW)])

    return sc_sum


PACK_C = 2048
NBLK = (V + PACK_C - 1) // PACK_C
VL = NBLK * PACK_C


def _pack_body(x_ref, o_ref):
    x = x_ref[...]
    eye = (lax.broadcasted_iota(jnp.int32, (D, D), 0)
           == lax.broadcasted_iota(jnp.int32, (D, D), 1)).astype(jnp.float32)
    xt = lax.dot_general(x, eye, (((0,), (0,)), ((), ())),
                         precision=lax.Precision.HIGHEST,
                         preferred_element_type=jnp.float32)
    o_ref[:, 0:D] = xt[0:PACK_C // 2, :]
    o_ref[:, D:2 * D] = xt[PACK_C // 2:PACK_C, :]


def _pack_table(tableT):
    return pl.pallas_call(
        _pack_body,
        grid=(NBLK,),
        in_specs=[pl.BlockSpec((D, PACK_C), lambda i: (0, i))],
        out_specs=pl.BlockSpec((PACK_C // 2, 2 * D), lambda i: (i, 0)),
        out_shape=jax.ShapeDtypeStruct((VL // 2, 2 * D), jnp.float32),
    )(tableT)


def _sigma_body(t_ref, o_ref):
    t = t_ref[...]
    hi = t & ~(PACK_C - 1)
    off = t & (PACK_C - 1)
    p = off >> 10
    lo = off & (PACK_C // 2 - 1)
    o_ref[...] = hi | (lo << 1) | p


def _sigma_tokens(tokens):
    blk = 1024
    return pl.pallas_call(
        _sigma_body,
        grid=(B // blk,),
        in_specs=[pl.BlockSpec((blk, L), lambda i: (i, 0))],
        out_specs=pl.BlockSpec((blk, L), lambda i: (i, 0)),
        out_shape=jax.ShapeDtypeStruct((B, L), jnp.int32),
    )(tokens)


def _mlp_body(tok_ref, sums_ref, w1_ref, b1_ref, w2_ref, b2_ref, out_ref):
    tok = tok_ref[...]
    cnt = jnp.sum((tok != 0).astype(jnp.float32), axis=1, keepdims=True)
    pooled = sums_ref[...] / jnp.maximum(cnt, 1.0)
    h = jnp.dot(pooled, w1_ref[...], precision=lax.Precision.HIGHEST,
                preferred_element_type=jnp.float32) + b1_ref[...]
    h = jnp.maximum(h, 0.0)
    p = jnp.dot(h, w2_ref[...], precision=lax.Precision.HIGHEST,
                preferred_element_type=jnp.float32) + b2_ref[...]
    nrm = jnp.sqrt(jnp.sum(p * p, axis=1, keepdims=True))
    out_ref[...] = p / jnp.maximum(nrm, 1e-8)


def _mlp(tokens, sums, W1, b1, W2, b2):
    blk = 1024
    grid = (B // blk,)
    return pl.pallas_call(
        _mlp_body,
        grid=grid,
        in_specs=[
            pl.BlockSpec((blk, L), lambda i: (i, 0)),
            pl.BlockSpec((blk, D), lambda i: (i, 0)),
            pl.BlockSpec((D, O), lambda i: (0, 0)),
            pl.BlockSpec((1, O), lambda i: (0, 0)),
            pl.BlockSpec((O, O), lambda i: (0, 0)),
            pl.BlockSpec((1, O), lambda i: (0, 0)),
        ],
        out_specs=pl.BlockSpec((blk, O), lambda i: (i, 0)),
        out_shape=jax.ShapeDtypeStruct((B, O), jnp.float32),
    )(tokens, sums, W1, b1, W2, b2)


def kernel(tokens, table, W1, b1, W2, b2):
    tok2 = _sigma_tokens(tokens).reshape(2 * B, HALF)
    packed = _pack_table(table.T)
    lin = packed.reshape(VL, D)
    sums = _make_sc_sum()(tok2, lin)
    return _mlp(tokens, sums, W1, b1.reshape(1, O), W2, b2.reshape(1, O))


# speedup vs baseline: 1.2543x; 1.2543x over previous
import functools

import jax
import jax.numpy as jnp
from jax import lax
from jax.experimental import pallas as pl
from jax.experimental.pallas import tpu as pltpu
from jax.experimental.pallas import tpu_sc as plsc

V, D, O = 1000000, 64, 256
B, L = 4096, 200

NC, NS = 2, 16
NW = NC * NS
EPW = B // NW
HALF = L // 2


def _make_sc_sum():
    mesh = plsc.VectorSubcoreMesh(core_axis_name="c", subcore_axis_name="s")

    @functools.partial(
        pl.kernel,
        mesh=mesh,
        compiler_params=pltpu.CompilerParams(use_tc_tiling_on_sc=False),
        out_type=jax.ShapeDtypeStruct((B, D), jnp.float32),
        scratch_types=[
            pltpu.VMEM((2 * EPW, HALF), jnp.int32),
            pltpu.VMEM((HALF, D), jnp.float32),
            pltpu.VMEM((HALF, D), jnp.float32),
            pltpu.VMEM((HALF, D), jnp.float32),
            pltpu.VMEM((HALF, D), jnp.float32),
            pltpu.VMEM((EPW, D), jnp.float32),
            pltpu.SemaphoreType.DMA,
            pltpu.SemaphoreType.DMA,
        ],
    )
    def sc_sum(tok_hbm, table_hbm, out_hbm, tok_v, a0_v, a1_v, b0_v, b1_v,
               out_v, sem_a, sem_b):
        wid = lax.axis_index("s") * NC + lax.axis_index("c")
        pltpu.sync_copy(tok_hbm.at[pl.ds(wid * (2 * EPW), 2 * EPW)], tok_v)

        def issue(e, r0, r1, sem):
            pltpu.async_copy(table_hbm.at[tok_v.at[2 * e]], r0, sem)
            pltpu.async_copy(table_hbm.at[tok_v.at[2 * e + 1]], r1, sem)

        def drain(r0, r1, sem):
            pltpu.make_async_copy(table_hbm.at[tok_v.at[0]], r0, sem).wait()
            pltpu.make_async_copy(table_hbm.at[tok_v.at[0]], r1, sem).wait()

        def sumbuf(e, r0, r1):
            z = jnp.zeros((16,), jnp.float32)

            def rbody(i, acc):
                a0, a1, a2, a3 = acc
                r = 2 * i
                a0 = a0 + r0[r, pl.ds(0, 16)] + r1[r, pl.ds(0, 16)]
                a1 = a1 + r0[r, pl.ds(16, 16)] + r1[r, pl.ds(16, 16)]
                a2 = a2 + r0[r, pl.ds(32, 16)] + r1[r, pl.ds(32, 16)]
                a3 = a3 + r0[r, pl.ds(48, 16)] + r1[r, pl.ds(48, 16)]
                s = r + 1
                a0 = a0 + r0[s, pl.ds(0, 16)] + r1[s, pl.ds(0, 16)]
                a1 = a1 + r0[s, pl.ds(16, 16)] + r1[s, pl.ds(16, 16)]
                a2 = a2 + r0[s, pl.ds(32, 16)] + r1[s, pl.ds(32, 16)]
                a3 = a3 + r0[s, pl.ds(48, 16)] + r1[s, pl.ds(48, 16)]
                return (a0, a1, a2, a3)

            a0, a1, a2, a3 = lax.fori_loop(0, HALF // 2, rbody, (z, z, z, z))
            out_v[e, pl.ds(0, 16)] = a0
            out_v[e, pl.ds(16, 16)] = a1
            out_v[e, pl.ds(32, 16)] = a2
            out_v[e, pl.ds(48, 16)] = a3

        issue(0, a0_v, a1_v, sem_a)

        def pair(i, carry):
            issue(2 * i + 1, b0_v, b1_v, sem_b)
            drain(a0_v, a1_v, sem_a)
            sumbuf(2 * i, a0_v, a1_v)

            @pl.when(i < EPW // 2 - 1)
            def _():
                issue(2 * i + 2, a0_v, a1_v, sem_a)

            drain(b0_v, b1_v, sem_b)
            sumbuf(2 * i + 1, b0_v, b1_v)
            return carry

        lax.fori_loop(0, EPW // 2, pair, 0)
        pltpu.sync_copy(out_v, out_hbm.at[pl.ds(wid * EPW, EPW)])

    return sc_sum


PACK_C = 2048
NBLK = (V + PACK_C - 1) // PACK_C
VL = NBLK * PACK_C


def _pack_body(x_ref, o_ref):
    xt = jnp.swapaxes(x_ref[...], 0, 1)
    o_ref[:, 0:D] = xt[0:PACK_C // 2, :]
    o_ref[:, D:2 * D] = xt[PACK_C // 2:PACK_C, :]


def _pack_table(tableT):
    return pl.pallas_call(
        _pack_body,
        grid=(NBLK,),
        in_specs=[pl.BlockSpec((D, PACK_C), lambda i: (0, i))],
        out_specs=pl.BlockSpec((PACK_C // 2, 2 * D), lambda i: (i, 0)),
        out_shape=jax.ShapeDtypeStruct((VL // 2, 2 * D), jnp.float32),
    )(tableT)


def _sigma_body(t_ref, o_ref):
    t = t_ref[...]
    hi = t & ~(PACK_C - 1)
    off = t & (PACK_C - 1)
    p = off >> 10
    lo = off & (PACK_C // 2 - 1)
    o_ref[...] = hi | (lo << 1) | p


def _sigma_tokens(tokens):
    blk = 1024
    return pl.pallas_call(
        _sigma_body,
        grid=(B // blk,),
        in_specs=[pl.BlockSpec((blk, L), lambda i: (i, 0))],
        out_specs=pl.BlockSpec((blk, L), lambda i: (i, 0)),
        out_shape=jax.ShapeDtypeStruct((B, L), jnp.int32),
    )(tokens)


def _mlp_body(tok_ref, sums_ref, w1_ref, b1_ref, w2_ref, b2_ref, out_ref):
    tok = tok_ref[...]
    cnt = jnp.sum((tok != 0).astype(jnp.float32), axis=1, keepdims=True)
    pooled = sums_ref[...] / jnp.maximum(cnt, 1.0)
    h = jnp.dot(pooled, w1_ref[...], precision=lax.Precision.HIGHEST,
                preferred_element_type=jnp.float32) + b1_ref[...]
    h = jnp.maximum(h, 0.0)
    p = jnp.dot(h, w2_ref[...], precision=lax.Precision.HIGHEST,
                preferred_element_type=jnp.float32) + b2_ref[...]
    nrm = jnp.sqrt(jnp.sum(p * p, axis=1, keepdims=True))
    out_ref[...] = p / jnp.maximum(nrm, 1e-8)


def _mlp(tokens, sums, W1, b1, W2, b2):
    blk = 1024
    grid = (B // blk,)
    return pl.pallas_call(
        _mlp_body,
        grid=grid,
        in_specs=[
            pl.BlockSpec((blk, L), lambda i: (i, 0)),
            pl.BlockSpec((blk, D), lambda i: (i, 0)),
            pl.BlockSpec((D, O), lambda i: (0, 0)),
            pl.BlockSpec((1, O), lambda i: (0, 0)),
            pl.BlockSpec((O, O), lambda i: (0, 0)),
            pl.BlockSpec((1, O), lambda i: (0, 0)),
        ],
        out_specs=pl.BlockSpec((blk, O), lambda i: (i, 0)),
        out_shape=jax.ShapeDtypeStruct((B, O), jnp.float32),
    )(tokens, sums, W1, b1, W2, b2)


def kernel(tokens, table, W1, b1, W2, b2):
    tok2 = _sigma_tokens(tokens).reshape(2 * B, HALF)
    packed = _pack_table(table.T)
    lin = packed.reshape(VL, D)
    sums = _make_sc_sum()(tok2, lin)
    return _mlp(tokens, sums, W1, b1.reshape(1, O), W2, b2.reshape(1, O))


# speedup vs baseline: 1.3244x; 1.0558x over previous
import functools

import jax
import jax.numpy as jnp
from jax import lax
from jax.experimental import pallas as pl
from jax.experimental.pallas import tpu as pltpu
from jax.experimental.pallas import tpu_sc as plsc

V, D, O = 1000000, 64, 256
B, L = 4096, 200

NC, NS = 2, 16
NW = NC * NS
EPW = B // NW
HALF = L // 2


def _make_sc_sum():
    mesh = plsc.VectorSubcoreMesh(core_axis_name="c", subcore_axis_name="s")

    @functools.partial(
        pl.kernel,
        mesh=mesh,
        compiler_params=pltpu.CompilerParams(use_tc_tiling_on_sc=False,
                                             needs_layout_passes=False),
        out_type=jax.ShapeDtypeStruct((B, D), jnp.float32),
        scratch_types=[
            pltpu.VMEM((2 * EPW, HALF), jnp.int32),
            pltpu.VMEM((HALF, D // 2), jnp.float32),
            pltpu.VMEM((HALF, D // 2), jnp.float32),
            pltpu.VMEM((HALF, D // 2), jnp.float32),
            pltpu.VMEM((HALF, D // 2), jnp.float32),
            pltpu.VMEM((EPW, D), jnp.float32),
            pltpu.SemaphoreType.DMA,
            pltpu.SemaphoreType.DMA,
        ],
    )
    def sc_sum(tok_hbm, table_hbm, out_hbm, tok_v, a0_v, a1_v, b0_v, b1_v,
               out_v, sem_a, sem_b):
        wid = lax.axis_index("s") * NC + lax.axis_index("c")
        pltpu.sync_copy(tok_hbm.at[pl.ds(wid * (2 * EPW), 2 * EPW)], tok_v)

        def issue(e, r0, r1, sem):
            pltpu.async_copy(table_hbm.at[tok_v.at[2 * e]], r0, sem)
            pltpu.async_copy(table_hbm.at[tok_v.at[2 * e + 1]], r1, sem)

        def drain(r0, r1, sem):
            pltpu.make_async_copy(table_hbm.at[tok_v.at[0]], r0, sem).wait()
            pltpu.make_async_copy(table_hbm.at[tok_v.at[0]], r1, sem).wait()

        def sumbuf(e, r0, r1):
            z = jnp.zeros((16,), jnp.float32)
            fmt = plsc.PackFormat.INTERLEAVED

            def rbody(i, acc):
                a0, a1, a2, a3 = acc
                for r in (2 * i, 2 * i + 1):
                    for buf in (r0, r1):
                        lo0, hi0 = plsc.unpack(
                            plsc.bitcast(buf[r, pl.ds(0, 16)], jnp.bfloat16),
                            format=fmt, preferred_element_type=jnp.float32)
                        lo1, hi1 = plsc.unpack(
                            plsc.bitcast(buf[r, pl.ds(16, 16)], jnp.bfloat16),
                            format=fmt, preferred_element_type=jnp.float32)
                        a0 = a0 + lo0
                        a1 = a1 + lo1
                        a2 = a2 + hi0
                        a3 = a3 + hi1
                return (a0, a1, a2, a3)

            a0, a1, a2, a3 = lax.fori_loop(0, HALF // 2, rbody, (z, z, z, z))
            out_v[e, pl.ds(0, 16)] = a0
            out_v[e, pl.ds(16, 16)] = a1
            out_v[e, pl.ds(32, 16)] = a2
            out_v[e, pl.ds(48, 16)] = a3

        issue(0, a0_v, a1_v, sem_a)

        def pair(i, carry):
            issue(2 * i + 1, b0_v, b1_v, sem_b)
            drain(a0_v, a1_v, sem_a)
            sumbuf(2 * i, a0_v, a1_v)

            @pl.when(i < EPW // 2 - 1)
            def _():
                issue(2 * i + 2, a0_v, a1_v, sem_a)

            drain(b0_v, b1_v, sem_b)
            sumbuf(2 * i + 1, b0_v, b1_v)
            return carry

        lax.fori_loop(0, EPW // 2, pair, 0)
        pltpu.sync_copy(out_v, out_hbm.at[pl.ds(wid * EPW, EPW)])

    return sc_sum


PACK_C = 2048
NBLK = (V + PACK_C - 1) // PACK_C
VL = NBLK * PACK_C


def _pack_body(x_ref, o_ref):
    x = x_ref[...]
    u = lax.bitcast_convert_type(x, jnp.uint32)
    r = u + jnp.uint32(0x7FFF) + ((u >> 16) & jnp.uint32(1))
    hi = r & jnp.uint32(0xFFFF0000)
    w = (hi[0:D // 2, :] >> 16) | hi[D // 2:D, :]
    wt = jnp.swapaxes(lax.bitcast_convert_type(w, jnp.float32), 0, 1)
    q = PACK_C // 4
    o_ref[:, 0:32] = wt[0 * q:1 * q]
    o_ref[:, 32:64] = wt[1 * q:2 * q]
    o_ref[:, 64:96] = wt[2 * q:3 * q]
    o_ref[:, 96:128] = wt[3 * q:4 * q]


def _pack_table(tableT):
    return pl.pallas_call(
        _pack_body,
        grid=(NBLK,),
        in_specs=[pl.BlockSpec((D, PACK_C), lambda i: (0, i))],
        out_specs=pl.BlockSpec((PACK_C // 4, 128), lambda i: (i, 0)),
        out_shape=jax.ShapeDtypeStruct((VL // 4, 128), jnp.float32),
    )(tableT)


def _sigma_body(t_ref, o_ref):
    t = t_ref[...]
    hi = t & ~(PACK_C - 1)
    off = t & (PACK_C - 1)
    q = off >> 9
    r = off & (PACK_C // 4 - 1)
    o_ref[...] = hi | (r << 2) | q


def _sigma_tokens(tokens):
    blk = 1024
    return pl.pallas_call(
        _sigma_body,
        grid=(B // blk,),
        in_specs=[pl.BlockSpec((blk, L), lambda i: (i, 0))],
        out_specs=pl.BlockSpec((blk, L), lambda i: (i, 0)),
        out_shape=jax.ShapeDtypeStruct((B, L), jnp.int32),
    )(tokens)


def _mlp_body(tok_ref, sums_ref, w1_ref, b1_ref, w2_ref, b2_ref, out_ref):
    tok = tok_ref[...]
    cnt = jnp.sum((tok != 0).astype(jnp.float32), axis=1, keepdims=True)
    pooled = sums_ref[...] / jnp.maximum(cnt, 1.0)
    h = jnp.dot(pooled, w1_ref[...], precision=lax.Precision.HIGHEST,
                preferred_element_type=jnp.float32) + b1_ref[...]
    h = jnp.maximum(h, 0.0)
    p = jnp.dot(h, w2_ref[...], precision=lax.Precision.HIGHEST,
                preferred_element_type=jnp.float32) + b2_ref[...]
    nrm = jnp.sqrt(jnp.sum(p * p, axis=1, keepdims=True))
    out_ref[...] = p / jnp.maximum(nrm, 1e-8)


def _mlp(tokens, sums, W1, b1, W2, b2):
    blk = 1024
    grid = (B // blk,)
    return pl.pallas_call(
        _mlp_body,
        grid=grid,
        in_specs=[
            pl.BlockSpec((blk, L), lambda i: (i, 0)),
            pl.BlockSpec((blk, D), lambda i: (i, 0)),
            pl.BlockSpec((D, O), lambda i: (0, 0)),
            pl.BlockSpec((1, O), lambda i: (0, 0)),
            pl.BlockSpec((O, O), lambda i: (0, 0)),
            pl.BlockSpec((1, O), lambda i: (0, 0)),
        ],
        out_specs=pl.BlockSpec((blk, O), lambda i: (i, 0)),
        out_shape=jax.ShapeDtypeStruct((B, O), jnp.float32),
    )(tokens, sums, W1, b1, W2, b2)


def kernel(tokens, table, W1, b1, W2, b2):
    tok2 = _sigma_tokens(tokens).reshape(2 * B, HALF)
    packed = _pack_table(table.T)
    lin = packed.reshape(VL, D // 2)
    sums = _make_sc_sum()(tok2, lin)
    return _mlp(tokens, sums, W1, b1.reshape(1, O), W2, b2.reshape(1, O))


# speedup vs baseline: 1.9559x; 1.4768x over previous
import functools

import jax
import jax.numpy as jnp
from jax import lax
from jax.experimental import pallas as pl
from jax.experimental.pallas import tpu as pltpu
from jax.experimental.pallas import tpu_sc as plsc

V, D, O = 1000000, 64, 256
B, L = 4096, 200

NC, NS = 2, 16
NW = NC * NS
EPW = B // NW
HALF = L // 2


def _make_sc_sum():
    mesh = plsc.VectorSubcoreMesh(core_axis_name="c", subcore_axis_name="s")

    @functools.partial(
        pl.kernel,
        mesh=mesh,
        compiler_params=pltpu.CompilerParams(use_tc_tiling_on_sc=False,
                                             needs_layout_passes=False),
        out_type=jax.ShapeDtypeStruct((B, D), jnp.float32),
        scratch_types=[
            pltpu.VMEM((2 * EPW, HALF), jnp.int32),
            pltpu.VMEM((HALF, D // 2), jnp.float32),
            pltpu.VMEM((HALF, D // 2), jnp.float32),
            pltpu.VMEM((HALF, D // 2), jnp.float32),
            pltpu.VMEM((HALF, D // 2), jnp.float32),
            pltpu.VMEM((EPW, D), jnp.float32),
            pltpu.SemaphoreType.DMA,
            pltpu.SemaphoreType.DMA,
        ],
    )
    def sc_sum(tok_hbm, table_hbm, out_hbm, tok_v, a0_v, a1_v, b0_v, b1_v,
               out_v, sem_a, sem_b):
        wid = lax.axis_index("s") * NC + lax.axis_index("c")
        pltpu.sync_copy(tok_hbm.at[pl.ds(wid * (2 * EPW), 2 * EPW)], tok_v)

        def issue(e, r0, r1, sem):
            pltpu.async_copy(table_hbm.at[tok_v.at[2 * e]], r0, sem)
            pltpu.async_copy(table_hbm.at[tok_v.at[2 * e + 1]], r1, sem)

        def drain(r0, r1, sem):
            pltpu.make_async_copy(table_hbm.at[tok_v.at[0]], r0, sem).wait()
            pltpu.make_async_copy(table_hbm.at[tok_v.at[0]], r1, sem).wait()

        def sumbuf(e, r0, r1):
            z = jnp.zeros((16,), jnp.float32)
            fmt = plsc.PackFormat.INTERLEAVED

            def rbody(i, acc):
                a0, a1, a2, a3 = acc
                for r in (2 * i, 2 * i + 1):
                    for buf in (r0, r1):
                        lo0, hi0 = plsc.unpack(
                            plsc.bitcast(buf[r, pl.ds(0, 16)], jnp.bfloat16),
                            format=fmt, preferred_element_type=jnp.float32)
                        lo1, hi1 = plsc.unpack(
                            plsc.bitcast(buf[r, pl.ds(16, 16)], jnp.bfloat16),
                            format=fmt, preferred_element_type=jnp.float32)
                        a0 = a0 + lo0
                        a1 = a1 + lo1
                        a2 = a2 + hi0
                        a3 = a3 + hi1
                return (a0, a1, a2, a3)

            a0, a1, a2, a3 = lax.fori_loop(0, HALF // 2, rbody, (z, z, z, z))
            out_v[e, pl.ds(0, 16)] = a0
            out_v[e, pl.ds(16, 16)] = a1
            out_v[e, pl.ds(32, 16)] = a2
            out_v[e, pl.ds(48, 16)] = a3

        issue(0, a0_v, a1_v, sem_a)

        def pair(i, carry):
            issue(2 * i + 1, b0_v, b1_v, sem_b)
            drain(a0_v, a1_v, sem_a)
            sumbuf(2 * i, a0_v, a1_v)

            @pl.when(i < EPW // 2 - 1)
            def _():
                issue(2 * i + 2, a0_v, a1_v, sem_a)

            drain(b0_v, b1_v, sem_b)
            sumbuf(2 * i + 1, b0_v, b1_v)
            return carry

        lax.fori_loop(0, EPW // 2, pair, 0)
        pltpu.sync_copy(out_v, out_hbm.at[pl.ds(wid * EPW, EPW)])

    return sc_sum


PACK_C = 8192
PACK_Q = PACK_C // 4
PACK_QSH = PACK_Q.bit_length() - 1
NBLK = (V + PACK_C - 1) // PACK_C
VL = NBLK * PACK_C


def _pack_body(x_ref, o_ref):
    x = x_ref[...]
    u = lax.bitcast_convert_type(x, jnp.uint32)
    r = u + jnp.uint32(0x7FFF) + ((u >> 16) & jnp.uint32(1))
    hi = r & jnp.uint32(0xFFFF0000)
    w = (hi[0:D // 2, :] >> 16) | hi[D // 2:D, :]
    wt = jnp.swapaxes(lax.bitcast_convert_type(w, jnp.float32), 0, 1)
    q = PACK_C // 4
    o_ref[:, 0:32] = wt[0 * q:1 * q]
    o_ref[:, 32:64] = wt[1 * q:2 * q]
    o_ref[:, 64:96] = wt[2 * q:3 * q]
    o_ref[:, 96:128] = wt[3 * q:4 * q]


def _pack_table(tableT):
    return pl.pallas_call(
        _pack_body,
        grid=(NBLK,),
        in_specs=[pl.BlockSpec((D, PACK_C), lambda i: (0, i))],
        out_specs=pl.BlockSpec((PACK_C // 4, 128), lambda i: (i, 0)),
        out_shape=jax.ShapeDtypeStruct((VL // 4, 128), jnp.float32),
    )(tableT)


def _sigma_body(t_ref, o_ref):
    t = t_ref[...]
    hi = t & ~(PACK_C - 1)
    off = t & (PACK_C - 1)
    q = off >> PACK_QSH
    r = off & (PACK_Q - 1)
    o_ref[...] = hi | (r << 2) | q


def _sigma_tokens(tokens):
    blk = 1024
    return pl.pallas_call(
        _sigma_body,
        grid=(B // blk,),
        in_specs=[pl.BlockSpec((blk, L), lambda i: (i, 0))],
        out_specs=pl.BlockSpec((blk, L), lambda i: (i, 0)),
        out_shape=jax.ShapeDtypeStruct((B, L), jnp.int32),
    )(tokens)


def _mlp_body(tok_ref, sums_ref, w1_ref, b1_ref, w2_ref, b2_ref, out_ref):
    tok = tok_ref[...]
    cnt = jnp.sum((tok != 0).astype(jnp.float32), axis=1, keepdims=True)
    pooled = sums_ref[...] / jnp.maximum(cnt, 1.0)
    h = jnp.dot(pooled, w1_ref[...], precision=lax.Precision.HIGHEST,
                preferred_element_type=jnp.float32) + b1_ref[...]
    h = jnp.maximum(h, 0.0)
    p = jnp.dot(h, w2_ref[...], precision=lax.Precision.HIGHEST,
                preferred_element_type=jnp.float32) + b2_ref[...]
    nrm = jnp.sqrt(jnp.sum(p * p, axis=1, keepdims=True))
    out_ref[...] = p / jnp.maximum(nrm, 1e-8)


def _mlp(tokens, sums, W1, b1, W2, b2):
    blk = 1024
    grid = (B // blk,)
    return pl.pallas_call(
        _mlp_body,
        grid=grid,
        in_specs=[
            pl.BlockSpec((blk, L), lambda i: (i, 0)),
            pl.BlockSpec((blk, D), lambda i: (i, 0)),
            pl.BlockSpec((D, O), lambda i: (0, 0)),
            pl.BlockSpec((1, O), lambda i: (0, 0)),
            pl.BlockSpec((O, O), lambda i: (0, 0)),
            pl.BlockSpec((1, O), lambda i: (0, 0)),
        ],
        out_specs=pl.BlockSpec((blk, O), lambda i: (i, 0)),
        out_shape=jax.ShapeDtypeStruct((B, O), jnp.float32),
    )(tokens, sums, W1, b1, W2, b2)


def kernel(tokens, table, W1, b1, W2, b2):
    tok2 = _sigma_tokens(tokens).reshape(2 * B, HALF)
    packed = _pack_table(table.T)
    lin = packed.reshape(VL, D // 2)
    sums = _make_sc_sum()(tok2, lin)
    return _mlp(tokens, sums, W1, b1.reshape(1, O), W2, b2.reshape(1, O))


# speedup vs baseline: 2.0081x; 1.0267x over previous
import functools

import jax
import jax.numpy as jnp
from jax import lax
from jax.experimental import pallas as pl
from jax.experimental.pallas import tpu as pltpu
from jax.experimental.pallas import tpu_sc as plsc

V, D, O = 1000000, 64, 256
B, L = 4096, 200

NC, NS = 2, 16
NW = NC * NS
EPW = B // NW
HALF = L // 2


def _make_sc_sum():
    mesh = plsc.VectorSubcoreMesh(core_axis_name="c", subcore_axis_name="s")

    @functools.partial(
        pl.kernel,
        mesh=mesh,
        compiler_params=pltpu.CompilerParams(use_tc_tiling_on_sc=False,
                                             needs_layout_passes=False),
        out_type=jax.ShapeDtypeStruct((B, D), jnp.float32),
        scratch_types=[
            pltpu.VMEM((2 * EPW, HALF), jnp.int32),
            pltpu.VMEM((HALF, D // 2), jnp.float32),
            pltpu.VMEM((HALF, D // 2), jnp.float32),
            pltpu.VMEM((HALF, D // 2), jnp.float32),
            pltpu.VMEM((HALF, D // 2), jnp.float32),
            pltpu.VMEM((EPW, D), jnp.float32),
            pltpu.SemaphoreType.DMA,
            pltpu.SemaphoreType.DMA,
        ],
    )
    def sc_sum(tok_hbm, table_hbm, out_hbm, tok_v, a0_v, a1_v, b0_v, b1_v,
               out_v, sem_a, sem_b):
        wid = lax.axis_index("s") * NC + lax.axis_index("c")
        pltpu.sync_copy(tok_hbm.at[pl.ds(wid * (2 * EPW), 2 * EPW)], tok_v)

        def issue(e, r0, r1, sem):
            pltpu.async_copy(table_hbm.at[tok_v.at[2 * e]], r0, sem)
            pltpu.async_copy(table_hbm.at[tok_v.at[2 * e + 1]], r1, sem)

        def drain(r0, r1, sem):
            pltpu.make_async_copy(table_hbm.at[tok_v.at[0]], r0, sem).wait()
            pltpu.make_async_copy(table_hbm.at[tok_v.at[0]], r1, sem).wait()

        def sumbuf(e, r0, r1):
            z = jnp.zeros((16,), jnp.float32)
            fmt = plsc.PackFormat.INTERLEAVED

            def rbody(i, acc):
                a0, a1, a2, a3 = acc
                for r in (2 * i, 2 * i + 1):
                    for buf in (r0, r1):
                        lo0, hi0 = plsc.unpack(
                            plsc.bitcast(buf[r, pl.ds(0, 16)], jnp.bfloat16),
                            format=fmt, preferred_element_type=jnp.float32)
                        lo1, hi1 = plsc.unpack(
                            plsc.bitcast(buf[r, pl.ds(16, 16)], jnp.bfloat16),
                            format=fmt, preferred_element_type=jnp.float32)
                        a0 = a0 + lo0
                        a1 = a1 + lo1
                        a2 = a2 + hi0
                        a3 = a3 + hi1
                return (a0, a1, a2, a3)

            a0, a1, a2, a3 = lax.fori_loop(0, HALF // 2, rbody, (z, z, z, z))
            out_v[e, pl.ds(0, 16)] = a0
            out_v[e, pl.ds(16, 16)] = a1
            out_v[e, pl.ds(32, 16)] = a2
            out_v[e, pl.ds(48, 16)] = a3

        issue(0, a0_v, a1_v, sem_a)

        def pair(i, carry):
            issue(2 * i + 1, b0_v, b1_v, sem_b)
            drain(a0_v, a1_v, sem_a)
            sumbuf(2 * i, a0_v, a1_v)

            @pl.when(i < EPW // 2 - 1)
            def _():
                issue(2 * i + 2, a0_v, a1_v, sem_a)

            drain(b0_v, b1_v, sem_b)
            sumbuf(2 * i + 1, b0_v, b1_v)
            return carry

        lax.fori_loop(0, EPW // 2, pair, 0)
        pltpu.sync_copy(out_v, out_hbm.at[pl.ds(wid * EPW, EPW)])

    return sc_sum


PACK_C = 16384
PACK_Q = PACK_C // 4
PACK_QSH = PACK_Q.bit_length() - 1
NBLK = (V + PACK_C - 1) // PACK_C
VL = NBLK * PACK_C


def _pack_body(x_ref, o_ref):
    x = x_ref[...]
    u = lax.bitcast_convert_type(x, jnp.uint32)
    r = u + jnp.uint32(0x7FFF) + ((u >> 16) & jnp.uint32(1))
    hi = r & jnp.uint32(0xFFFF0000)
    w = (hi[0:D // 2, :] >> 16) | hi[D // 2:D, :]
    wt = jnp.swapaxes(lax.bitcast_convert_type(w, jnp.float32), 0, 1)
    q = PACK_C // 4
    o_ref[:, 0:32] = wt[0 * q:1 * q]
    o_ref[:, 32:64] = wt[1 * q:2 * q]
    o_ref[:, 64:96] = wt[2 * q:3 * q]
    o_ref[:, 96:128] = wt[3 * q:4 * q]


def _pack_table(tableT):
    return pl.pallas_call(
        _pack_body,
        grid=(NBLK,),
        in_specs=[pl.BlockSpec((D, PACK_C), lambda i: (0, i))],
        out_specs=pl.BlockSpec((PACK_C // 4, 128), lambda i: (i, 0)),
        out_shape=jax.ShapeDtypeStruct((VL // 4, 128), jnp.float32),
    )(tableT)


def _sigma_body(t_ref, o_ref):
    t = t_ref[...]
    hi = t & ~(PACK_C - 1)
    off = t & (PACK_C - 1)
    q = off >> PACK_QSH
    r = off & (PACK_Q - 1)
    o_ref[...] = hi | (r << 2) | q


def _sigma_tokens(tokens):
    blk = 1024
    return pl.pallas_call(
        _sigma_body,
        grid=(B // blk,),
        in_specs=[pl.BlockSpec((blk, L), lambda i: (i, 0))],
        out_specs=pl.BlockSpec((blk, L), lambda i: (i, 0)),
        out_shape=jax.ShapeDtypeStruct((B, L), jnp.int32),
    )(tokens)


def _mlp_body(tok_ref, sums_ref, w1_ref, b1_ref, w2_ref, b2_ref, out_ref):
    tok = tok_ref[...]
    cnt = jnp.sum((tok != 0).astype(jnp.float32), axis=1, keepdims=True)
    pooled = sums_ref[...] / jnp.maximum(cnt, 1.0)
    h = jnp.dot(pooled, w1_ref[...], precision=lax.Precision.HIGHEST,
                preferred_element_type=jnp.float32) + b1_ref[...]
    h = jnp.maximum(h, 0.0)
    p = jnp.dot(h, w2_ref[...], precision=lax.Precision.HIGHEST,
                preferred_element_type=jnp.float32) + b2_ref[...]
    nrm = jnp.sqrt(jnp.sum(p * p, axis=1, keepdims=True))
    out_ref[...] = p / jnp.maximum(nrm, 1e-8)


def _mlp(tokens, sums, W1, b1, W2, b2):
    blk = 1024
    grid = (B // blk,)
    return pl.pallas_call(
        _mlp_body,
        grid=grid,
        in_specs=[
            pl.BlockSpec((blk, L), lambda i: (i, 0)),
            pl.BlockSpec((blk, D), lambda i: (i, 0)),
            pl.BlockSpec((D, O), lambda i: (0, 0)),
            pl.BlockSpec((1, O), lambda i: (0, 0)),
            pl.BlockSpec((O, O), lambda i: (0, 0)),
            pl.BlockSpec((1, O), lambda i: (0, 0)),
        ],
        out_specs=pl.BlockSpec((blk, O), lambda i: (i, 0)),
        out_shape=jax.ShapeDtypeStruct((B, O), jnp.float32),
    )(tokens, sums, W1, b1, W2, b2)


def kernel(tokens, table, W1, b1, W2, b2):
    tok2 = _sigma_tokens(tokens).reshape(2 * B, HALF)
    packed = _pack_table(table.T)
    lin = packed.reshape(VL, D // 2)
    sums = _make_sc_sum()(tok2, lin)
    return _mlp(tokens, sums, W1, b1.reshape(1, O), W2, b2.reshape(1, O))


# speedup vs baseline: 2.0119x; 1.0019x over previous
import functools

import jax
import jax.numpy as jnp
from jax import lax
from jax.experimental import pallas as pl
from jax.experimental.pallas import tpu as pltpu
from jax.experimental.pallas import tpu_sc as plsc

V, D, O = 1000000, 64, 256
B, L = 4096, 200

NC, NS = 2, 16
NW = NC * NS
EPW = B // NW
HALF = L // 2


def _make_sc_sum():
    mesh = plsc.VectorSubcoreMesh(core_axis_name="c", subcore_axis_name="s")

    @functools.partial(
        pl.kernel,
        mesh=mesh,
        compiler_params=pltpu.CompilerParams(use_tc_tiling_on_sc=False,
                                             needs_layout_passes=False),
        out_type=jax.ShapeDtypeStruct((B, D), jnp.float32),
        scratch_types=[
            pltpu.VMEM((2 * EPW, HALF), jnp.int32),
            pltpu.VMEM((HALF, D // 2), jnp.float32),
            pltpu.VMEM((HALF, D // 2), jnp.float32),
            pltpu.VMEM((HALF, D // 2), jnp.float32),
            pltpu.VMEM((HALF, D // 2), jnp.float32),
            pltpu.VMEM((EPW, D), jnp.float32),
            pltpu.SemaphoreType.DMA,
            pltpu.SemaphoreType.DMA,
        ],
    )
    def sc_sum(tok_hbm, table_hbm, out_hbm, tok_v, a0_v, a1_v, b0_v, b1_v,
               out_v, sem_a, sem_b):
        wid = lax.axis_index("s") * NC + lax.axis_index("c")
        pltpu.sync_copy(tok_hbm.at[pl.ds(wid * (2 * EPW), 2 * EPW)], tok_v)

        def issue(e, r0, r1, sem):
            pltpu.async_copy(table_hbm.at[tok_v.at[2 * e]], r0, sem)
            pltpu.async_copy(table_hbm.at[tok_v.at[2 * e + 1]], r1, sem)

        def drain(r0, r1, sem):
            pltpu.make_async_copy(table_hbm.at[tok_v.at[0]], r0, sem).wait()
            pltpu.make_async_copy(table_hbm.at[tok_v.at[0]], r1, sem).wait()

        def sumbuf(e, r0, r1):
            z = jnp.zeros((16,), jnp.float32)
            fmt = plsc.PackFormat.INTERLEAVED

            def rbody(i, acc):
                a0, a1, a2, a3 = acc
                for r in (2 * i, 2 * i + 1):
                    for buf in (r0, r1):
                        lo0, hi0 = plsc.unpack(
                            plsc.bitcast(buf[r, pl.ds(0, 16)], jnp.bfloat16),
                            format=fmt, preferred_element_type=jnp.float32)
                        lo1, hi1 = plsc.unpack(
                            plsc.bitcast(buf[r, pl.ds(16, 16)], jnp.bfloat16),
                            format=fmt, preferred_element_type=jnp.float32)
                        a0 = a0 + lo0
                        a1 = a1 + lo1
                        a2 = a2 + hi0
                        a3 = a3 + hi1
                return (a0, a1, a2, a3)

            a0, a1, a2, a3 = lax.fori_loop(0, HALF // 2, rbody, (z, z, z, z))
            out_v[e, pl.ds(0, 16)] = a0
            out_v[e, pl.ds(16, 16)] = a1
            out_v[e, pl.ds(32, 16)] = a2
            out_v[e, pl.ds(48, 16)] = a3

        issue(0, a0_v, a1_v, sem_a)

        def pair(i, carry):
            issue(2 * i + 1, b0_v, b1_v, sem_b)
            drain(a0_v, a1_v, sem_a)
            sumbuf(2 * i, a0_v, a1_v)

            @pl.when(i < EPW // 2 - 1)
            def _():
                issue(2 * i + 2, a0_v, a1_v, sem_a)

            drain(b0_v, b1_v, sem_b)
            sumbuf(2 * i + 1, b0_v, b1_v)
            return carry

        lax.fori_loop(0, EPW // 2, pair, 0)
        pltpu.sync_copy(out_v, out_hbm.at[pl.ds(wid * EPW, EPW)])

    return sc_sum


PACK_C = 32768
PACK_Q = PACK_C // 4
PACK_QSH = PACK_Q.bit_length() - 1
NBLK = (V + PACK_C - 1) // PACK_C
VL = NBLK * PACK_C


def _pack_body(x_ref, o_ref):
    x = x_ref[...]
    u = lax.bitcast_convert_type(x, jnp.uint32)
    r = u + jnp.uint32(0x7FFF) + ((u >> 16) & jnp.uint32(1))
    hi = r & jnp.uint32(0xFFFF0000)
    w = (hi[0:D // 2, :] >> 16) | hi[D // 2:D, :]
    wt = jnp.swapaxes(lax.bitcast_convert_type(w, jnp.float32), 0, 1)
    q = PACK_C // 4
    o_ref[:, 0:32] = wt[0 * q:1 * q]
    o_ref[:, 32:64] = wt[1 * q:2 * q]
    o_ref[:, 64:96] = wt[2 * q:3 * q]
    o_ref[:, 96:128] = wt[3 * q:4 * q]


def _pack_table(tableT):
    return pl.pallas_call(
        _pack_body,
        grid=(NBLK,),
        in_specs=[pl.BlockSpec((D, PACK_C), lambda i: (0, i))],
        out_specs=pl.BlockSpec((PACK_C // 4, 128), lambda i: (i, 0)),
        out_shape=jax.ShapeDtypeStruct((VL // 4, 128), jnp.float32),
    )(tableT)


def _sigma_body(t_ref, o_ref):
    t = t_ref[...]
    hi = t & ~(PACK_C - 1)
    off = t & (PACK_C - 1)
    q = off >> PACK_QSH
    r = off & (PACK_Q - 1)
    o_ref[...] = hi | (r << 2) | q


def _sigma_tokens(tokens):
    blk = 1024
    return pl.pallas_call(
        _sigma_body,
        grid=(B // blk,),
        in_specs=[pl.BlockSpec((blk, L), lambda i: (i, 0))],
        out_specs=pl.BlockSpec((blk, L), lambda i: (i, 0)),
        out_shape=jax.ShapeDtypeStruct((B, L), jnp.int32),
    )(tokens)


def _mlp_body(tok_ref, sums_ref, w1_ref, b1_ref, w2_ref, b2_ref, out_ref):
    tok = tok_ref[...]
    cnt = jnp.sum((tok != 0).astype(jnp.float32), axis=1, keepdims=True)
    pooled = sums_ref[...] / jnp.maximum(cnt, 1.0)
    h = jnp.dot(pooled, w1_ref[...], precision=lax.Precision.HIGHEST,
                preferred_element_type=jnp.float32) + b1_ref[...]
    h = jnp.maximum(h, 0.0)
    p = jnp.dot(h, w2_ref[...], precision=lax.Precision.HIGHEST,
                preferred_element_type=jnp.float32) + b2_ref[...]
    nrm = jnp.sqrt(jnp.sum(p * p, axis=1, keepdims=True))
    out_ref[...] = p / jnp.maximum(nrm, 1e-8)


def _mlp(tokens, sums, W1, b1, W2, b2):
    blk = 1024
    grid = (B // blk,)
    return pl.pallas_call(
        _mlp_body,
        grid=grid,
        in_specs=[
            pl.BlockSpec((blk, L), lambda i: (i, 0)),
            pl.BlockSpec((blk, D), lambda i: (i, 0)),
            pl.BlockSpec((D, O), lambda i: (0, 0)),
            pl.BlockSpec((1, O), lambda i: (0, 0)),
            pl.BlockSpec((O, O), lambda i: (0, 0)),
            pl.BlockSpec((1, O), lambda i: (0, 0)),
        ],
        out_specs=pl.BlockSpec((blk, O), lambda i: (i, 0)),
        out_shape=jax.ShapeDtypeStruct((B, O), jnp.float32),
    )(tokens, sums, W1, b1, W2, b2)


def kernel(tokens, table, W1, b1, W2, b2):
    tok2 = _sigma_tokens(tokens).reshape(2 * B, HALF)
    packed = _pack_table(table.T)
    lin = packed.reshape(VL, D // 2)
    sums = _make_sc_sum()(tok2, lin)
    return _mlp(tokens, sums, W1, b1.reshape(1, O), W2, b2.reshape(1, O))
